# baseline (speedup 1.0000x reference)
import functools

import jax
import jax.numpy as jnp
from jax import lax
from jax.experimental import pallas as pl
from jax.experimental.pallas import tpu as pltpu
from jax.experimental.pallas import tpu_sc as plsc

B = 16384
F = 26
V = 100000
D = 32

NC = 2
NS = 16
NW = NC * NS
PLANES = F * D
PER_W = PLANES // NW
L = 16
H0N = 50048
H1OFF = 49920
H1N = 50048
TOFF = V - 128
TN = 128
H1B = H1N + TN
IH = B // 2
GRP = IH // L

_mesh = plsc.VectorSubcoreMesh(core_axis_name="c", subcore_axis_name="s")


@functools.partial(
    pl.kernel,
    out_type=jax.ShapeDtypeStruct((F, D, B), jnp.float32),
    mesh=_mesh,
    scratch_types=[
        pltpu.VMEM((H0N,), jnp.float32),
        pltpu.VMEM((H1B,), jnp.float32),
        pltpu.VMEM((B,), jnp.int32),
        pltpu.VMEM((IH,), jnp.float32),
        pltpu.SemaphoreType.DMA,
        pltpu.SemaphoreType.DMA,
    ],
    compiler_params=pltpu.CompilerParams(needs_layout_passes=False),
)
def _plane_gather(x_t_hbm, w_t_hbm, tail_hbm, out_hbm,
                  h0, h1, idx_v, oh, sem0, sem1):
    wid = lax.axis_index("s") * NC + lax.axis_index("c")
    p0 = wid * PER_W

    def issue_h0(f, d):
        pltpu.async_copy(w_t_hbm.at[f, d].at[pl.ds(0, H0N)], h0, sem0)

    def issue_h1(f, d):
        pltpu.async_copy(
            w_t_hbm.at[f, d].at[pl.ds(H1OFF, H1N)],
            h1.at[pl.ds(0, H1N)], sem1)
        pltpu.async_copy(tail_hbm.at[f, d], h1.at[pl.ds(H1N, TN)], sem1)

    def wait_h0(f, d):
        pltpu.make_async_copy(
            w_t_hbm.at[f, d].at[pl.ds(0, H0N)], h0, sem0).wait()

    def wait_h1(f, d):
        pltpu.make_async_copy(
            w_t_hbm.at[f, d].at[pl.ds(H1OFF, H1N)],
            h1.at[pl.ds(0, H1N)], sem1).wait()
        pltpu.make_async_copy(
            tail_hbm.at[f, d], h1.at[pl.ds(H1N, TN)], sem1).wait()

    def sweep_lo(ih_base):
        def g(i, c):
            ii = idx_v[pl.ds(ih_base + i * L, L)]
            m = ii < H0N
            oh[pl.ds(i * L, L)] = plsc.load_gather(h0, [ii], mask=m)
            return c

        lax.fori_loop(0, GRP, g, 0, unroll=8)

    def sweep_hi(ih_base):
        def g(i, c):
            ii = idx_v[pl.ds(ih_base + i * L, L)]
            m = ii < H0N
            jj = ii - jnp.where(ii < H1OFF + H1N, H1OFF, H1OFF - 96)
            cur = oh[pl.ds(i * L, L)]
            g1 = plsc.load_gather(h1, [jj], mask=jnp.logical_not(m))
            oh[pl.ds(i * L, L)] = jnp.where(m, cur, g1)
            return c

        lax.fori_loop(0, GRP, g, 0, unroll=8)

    f_first = p0 // D
    d_first = p0 % D
    pltpu.sync_copy(x_t_hbm.at[f_first], idx_v)
    issue_h0(f_first, d_first)
    issue_h1(f_first, d_first)

    def plane(p, carry):
        pp = p0 + p
        f = pp // D
        d = pp % D
        pn = pp + 1
        fn = pn // D
        dn = pn % D

        @pl.when(jnp.logical_and(d == 0, p > 0))
        def _():
            pltpu.sync_copy(x_t_hbm.at[f], idx_v)

        wait_h0(f, d)
        sweep_lo(0)
        wait_h1(f, d)
        sweep_hi(0)
        pltpu.sync_copy(oh, out_hbm.at[f, d].at[pl.ds(0, IH)])

        sweep_lo(IH)

        @pl.when(p < PER_W - 1)
        def _():
            issue_h0(fn, dn)

        sweep_hi(IH)

        @pl.when(p < PER_W - 1)
        def _():
            issue_h1(fn, dn)

        pltpu.sync_copy(oh, out_hbm.at[f, d].at[pl.ds(IH, IH)])
        return carry

    lax.fori_loop(0, PER_W, plane, 0)


def kernel(x, W):
    w_t = W.transpose(0, 2, 1)
    w_tail = w_t[:, :, TOFF:V]
    out_t = _plane_gather(x.T.astype(jnp.int32), w_t, w_tail)
    return out_t.transpose(2, 0, 1)


# speedup vs baseline: 2.9043x; 2.9043x over previous
import functools

import jax
import jax.numpy as jnp
from jax import lax
from jax.experimental import pallas as pl
from jax.experimental.pallas import tpu as pltpu
from jax.experimental.pallas import tpu_sc as plsc

B = 16384
F = 26
V = 100000
D = 32

NC = 2
NS = 16
NW = NC * NS
PLANES = F * D
PER_W = PLANES // NW
L = 16
H0N = 50048
H1OFF = 49920
H1N = 50048
TOFF = V - 128
TN = 128
H1B = H1N + TN
IH = B // 2
GRP = IH // L

_mesh = plsc.VectorSubcoreMesh(core_axis_name="c", subcore_axis_name="s")


@functools.partial(
    pl.kernel,
    out_type=jax.ShapeDtypeStruct((F, D, B), jnp.float32),
    mesh=_mesh,
    scratch_types=[
        pltpu.VMEM((H0N,), jnp.float32),
        pltpu.VMEM((H1B,), jnp.float32),
        pltpu.VMEM((B,), jnp.int32),
        pltpu.VMEM((IH,), jnp.float32),
        pltpu.SemaphoreType.DMA,
        pltpu.SemaphoreType.DMA,
    ],
    compiler_params=pltpu.CompilerParams(needs_layout_passes=False),
)
def _plane_gather(x_t_hbm, w_t_hbm, tail_hbm, out_hbm,
                  h0, h1, idx_v, oh, sem0, sem1):
    wid = lax.axis_index("s") * NC + lax.axis_index("c")
    p0 = wid * PER_W

    def issue_h0(f, d):
        pltpu.async_copy(w_t_hbm.at[f, d].at[pl.ds(0, H0N)], h0, sem0)

    def issue_h1(f, d):
        pltpu.async_copy(
            w_t_hbm.at[f, d].at[pl.ds(H1OFF, H1N)],
            h1.at[pl.ds(0, H1N)], sem1)
        pltpu.async_copy(tail_hbm.at[f, d], h1.at[pl.ds(H1N, TN)], sem1)

    def wait_h0(f, d):
        pltpu.make_async_copy(
            w_t_hbm.at[f, d].at[pl.ds(0, H0N)], h0, sem0).wait()

    def wait_h1(f, d):
        pltpu.make_async_copy(
            w_t_hbm.at[f, d].at[pl.ds(H1OFF, H1N)],
            h1.at[pl.ds(0, H1N)], sem1).wait()
        pltpu.make_async_copy(
            tail_hbm.at[f, d], h1.at[pl.ds(H1N, TN)], sem1).wait()

    def sweep_lo(ih_base):
        @plsc.parallel_loop(0, GRP, 1, unroll=8)
        def _(i):
            ii = idx_v[pl.ds(ih_base + i * L, L)]
            m = ii < H0N
            oh[pl.ds(i * L, L)] = plsc.load_gather(h0, [ii], mask=m)

    def sweep_hi(ih_base):
        @plsc.parallel_loop(0, GRP, 1, unroll=8)
        def _(i):
            ii = idx_v[pl.ds(ih_base + i * L, L)]
            m = ii < H0N
            jj = ii - jnp.where(ii < H1OFF + H1N, H1OFF, H1OFF - 96)
            cur = oh[pl.ds(i * L, L)]
            g1 = plsc.load_gather(h1, [jj], mask=jnp.logical_not(m))
            oh[pl.ds(i * L, L)] = jnp.where(m, cur, g1)

    f_first = p0 // D
    d_first = p0 % D
    pltpu.sync_copy(x_t_hbm.at[f_first], idx_v)
    issue_h0(f_first, d_first)
    issue_h1(f_first, d_first)

    def plane(p, carry):
        pp = p0 + p
        f = pp // D
        d = pp % D
        pn = pp + 1
        fn = pn // D
        dn = pn % D

        @pl.when(jnp.logical_and(d == 0, p > 0))
        def _():
            pltpu.sync_copy(x_t_hbm.at[f], idx_v)

        wait_h0(f, d)
        sweep_lo(0)
        wait_h1(f, d)
        sweep_hi(0)
        pltpu.sync_copy(oh, out_hbm.at[f, d].at[pl.ds(0, IH)])

        sweep_lo(IH)

        @pl.when(p < PER_W - 1)
        def _():
            issue_h0(fn, dn)

        sweep_hi(IH)

        @pl.when(p < PER_W - 1)
        def _():
            issue_h1(fn, dn)

        pltpu.sync_copy(oh, out_hbm.at[f, d].at[pl.ds(IH, IH)])
        return carry

    lax.fori_loop(0, PER_W, plane, 0)


def kernel(x, W):
    w_t = W.transpose(0, 2, 1)
    w_tail = w_t[:, :, TOFF:V]
    out_t = _plane_gather(x.T.astype(jnp.int32), w_t, w_tail)
    return out_t.transpose(2, 0, 1)


# speedup vs baseline: 2.9473x; 1.0148x over previous
import functools

import jax
import jax.numpy as jnp
from jax import lax
from jax.experimental import pallas as pl
from jax.experimental.pallas import tpu as pltpu
from jax.experimental.pallas import tpu_sc as plsc

B = 16384
F = 26
V = 100000
D = 32

NC = 2
NS = 16
NW = NC * NS
PLANES = F * D
PER_W = PLANES // NW
L = 16
H0N = 50048
H1OFF = 49920
H1N = 50048
TOFF = V - 128
TN = 128
H1B = H1N + TN
IH = B // 2
GRP = IH // L

_mesh = plsc.VectorSubcoreMesh(core_axis_name="c", subcore_axis_name="s")


@functools.partial(
    pl.kernel,
    out_type=jax.ShapeDtypeStruct((F, D, B), jnp.float32),
    mesh=_mesh,
    scratch_types=[
        pltpu.VMEM((H0N,), jnp.float32),
        pltpu.VMEM((H1B,), jnp.float32),
        pltpu.VMEM((B,), jnp.int32),
        pltpu.VMEM((IH,), jnp.float32),
        pltpu.SemaphoreType.DMA,
        pltpu.SemaphoreType.DMA,
    ],
    compiler_params=pltpu.CompilerParams(needs_layout_passes=False),
)
def _plane_gather(x_t_hbm, w_t_hbm, tail_hbm, out_hbm,
                  h0, h1, idx_v, oh, sem0, sem1):
    wid = lax.axis_index("s") * NC + lax.axis_index("c")
    p0 = wid * PER_W

    def issue_h0(f, d):
        pltpu.async_copy(w_t_hbm.at[f, d].at[pl.ds(0, H0N)], h0, sem0)

    def issue_h1(f, d):
        pltpu.async_copy(
            w_t_hbm.at[f, d].at[pl.ds(H1OFF, H1N)],
            h1.at[pl.ds(0, H1N)], sem1)
        pltpu.async_copy(tail_hbm.at[f, d], h1.at[pl.ds(H1N, TN)], sem1)

    def wait_h0(f, d):
        pltpu.make_async_copy(
            w_t_hbm.at[f, d].at[pl.ds(0, H0N)], h0, sem0).wait()

    def wait_h1(f, d):
        pltpu.make_async_copy(
            w_t_hbm.at[f, d].at[pl.ds(H1OFF, H1N)],
            h1.at[pl.ds(0, H1N)], sem1).wait()
        pltpu.make_async_copy(
            tail_hbm.at[f, d], h1.at[pl.ds(H1N, TN)], sem1).wait()

    def sweep_lo(ih_base):
        @plsc.parallel_loop(0, GRP, 1, unroll=8)
        def _(i):
            ii = idx_v[pl.ds(ih_base + i * L, L)]
            m = ii < H0N
            oh[pl.ds(i * L, L)] = plsc.load_gather(h0, [ii], mask=m)

    def sweep_hi(ih_base):
        lane = lax.iota(jnp.int32, L)

        @plsc.parallel_loop(0, GRP, 1, unroll=8)
        def _(i):
            ii = idx_v[pl.ds(ih_base + i * L, L)]
            hi = ii >= H0N
            jj = ii - jnp.where(ii < H1OFF + H1N, H1OFF, H1OFF - 96)
            g1 = plsc.load_gather(h1, [jj], mask=hi)
            plsc.store_scatter(oh, [i * L + lane], g1, mask=hi)

    f_first = p0 // D
    d_first = p0 % D
    pltpu.sync_copy(x_t_hbm.at[f_first], idx_v)
    issue_h0(f_first, d_first)
    issue_h1(f_first, d_first)

    def plane(p, carry):
        pp = p0 + p
        f = pp // D
        d = pp % D
        pn = pp + 1
        fn = pn // D
        dn = pn % D

        @pl.when(jnp.logical_and(d == 0, p > 0))
        def _():
            pltpu.sync_copy(x_t_hbm.at[f], idx_v)

        wait_h0(f, d)
        sweep_lo(0)
        wait_h1(f, d)
        sweep_hi(0)
        pltpu.sync_copy(oh, out_hbm.at[f, d].at[pl.ds(0, IH)])

        sweep_lo(IH)

        @pl.when(p < PER_W - 1)
        def _():
            issue_h0(fn, dn)

        sweep_hi(IH)

        @pl.when(p < PER_W - 1)
        def _():
            issue_h1(fn, dn)

        pltpu.sync_copy(oh, out_hbm.at[f, d].at[pl.ds(IH, IH)])
        return carry

    lax.fori_loop(0, PER_W, plane, 0)


def kernel(x, W):
    w_t = W.transpose(0, 2, 1)
    w_tail = W[:, TOFF:V, :].transpose(0, 2, 1)
    out_t = _plane_gather(x.T.astype(jnp.int32), w_t, w_tail)
    return out_t.transpose(2, 0, 1)
